# baseline (speedup 1.0000x reference)
import functools

import jax
import jax.numpy as jnp
from jax import lax
from jax.experimental import pallas as pl
from jax.experimental.pallas import tpu as pltpu
from jax.experimental.pallas import tpu_sc as plsc

N_NODES = 10000
N_EDGES = 320000
EBLK = 128
NBLK = N_EDGES // EBLK
NC, NS = 2, 16
NPAIR = (NBLK // NS + 1) // 2 + 1
NPAIR_C = (NBLK // (NC * NS) + 1) // 2 + 1
R_MAIN = (N_NODES // NS) // 8 * 8
R_TAIL = N_NODES - NS * R_MAIN


def _sc_fused_body(node2_hbm, edge_hbm, eidx_hbm, z64_hbm,
                   z16_hbm, ones_hbm, ebw_hbm, hu2_hbm, hv2_hbm,
                   g_out, es_out, deg_out, eout_hbm,
                   g_sh, es_sh, deg_sh,
                   sidx0, sidx1, didx0, didx1, gidx0, gidx1,
                   rows0, rows1, a0, a1, b0, b1, erows_v, ones_v,
                   acc_w, nacc,
                   semg0, semg1, sema0, sema1, semb0, semb1):
    c = lax.axis_index("c")
    s = lax.axis_index("s")
    r0 = s * R_MAIN
    t0 = NS * R_MAIN
    WROWS = EBLK // 8

    pltpu.sync_copy(z64_hbm, g_sh.at[pl.ds(r0, R_MAIN)])
    pltpu.sync_copy(z16_hbm, es_sh.at[pl.ds(r0, R_MAIN)])
    pltpu.sync_copy(z16_hbm, deg_sh.at[pl.ds(r0, R_MAIN)])
    pltpu.sync_copy(ones_hbm, ones_v)

    @pl.when(s == 0)
    def _():
        pltpu.sync_copy(z64_hbm.at[pl.ds(0, R_TAIL)],
                        g_sh.at[pl.ds(t0, R_TAIL)])
        pltpu.sync_copy(z16_hbm.at[pl.ds(0, R_TAIL)],
                        es_sh.at[pl.ds(t0, R_TAIL)])
        pltpu.sync_copy(z16_hbm.at[pl.ds(0, R_TAIL)],
                        deg_sh.at[pl.ds(t0, R_TAIL)])

    plsc.subcore_barrier()

    def issue(tb, sidx, didx, gidx, rows, a_v, b_v, semg, sema, semb):
        j = tb * NS + s

        @pl.when(j < NBLK)
        def _():
            off = j * EBLK
            pltpu.sync_copy(eidx_hbm.at[0, pl.ds(off, EBLK)], sidx)
            pltpu.sync_copy(eidx_hbm.at[1, pl.ds(off, EBLK)], didx)
            for k in range(EBLK // 16):
                sl = pl.ds(k * 16, 16)
                gidx[sl] = sidx[sl] * 2 + c
            pltpu.make_async_copy(node2_hbm.at[gidx], rows, semg).start()

            @pl.when(lax.rem(j, 2) == c)
            def _():
                pltpu.make_async_copy(hu2_hbm.at[sidx], a_v, sema).start()
                pltpu.make_async_copy(hv2_hbm.at[didx], b_v, semb).start()

    def process(tb, sidx, didx, gidx, rows, a_v, b_v, semg, sema, semb):
        j = tb * NS + s

        @pl.when(j < NBLK)
        def _():
            off = j * EBLK
            pltpu.make_async_copy(node2_hbm.at[gidx], rows, semg).wait()
            pltpu.sync_copy(rows, g_sh.at[didx], add=True)

            @pl.when(lax.rem(j, 2) == c)
            def _():
                pltpu.sync_copy(edge_hbm.at[pl.ds(off, EBLK)], erows_v)
                pltpu.sync_copy(erows_v, es_sh.at[didx], add=True)
                pltpu.sync_copy(ebw_hbm.at[pl.ds(j * WROWS, WROWS)], acc_w)
                pltpu.make_async_copy(hu2_hbm.at[sidx], a_v, sema).wait()
                pltpu.make_async_copy(hv2_hbm.at[didx], b_v, semb).wait()

                def add_rows(q, carry2):
                    for u in range(8):
                        r = q * 8 + u
                        sl = pl.ds(u * 16, 16)
                        nacc[r] = acc_w[q, sl] + a_v[r] + b_v[r]
                    return carry2

                lax.fori_loop(0, WROWS, add_rows, 0)
                pltpu.sync_copy(nacc, eout_hbm.at[pl.ds(off, EBLK)])

            @pl.when(lax.rem(j, 2) != c)
            def _():
                pltpu.sync_copy(ones_v, deg_sh.at[didx], add=True)

    issue(0, sidx0, didx0, gidx0, rows0, a0, b0, semg0, sema0, semb0)

    def body(t, carry):
        tb = t * 2
        issue(tb + 1, sidx1, didx1, gidx1, rows1, a1, b1,
              semg1, sema1, semb1)
        process(tb, sidx0, didx0, gidx0, rows0, a0, b0,
                semg0, sema0, semb0)
        issue(tb + 2, sidx0, didx0, gidx0, rows0, a0, b0,
              semg0, sema0, semb0)
        process(tb + 1, sidx1, didx1, gidx1, rows1, a1, b1,
                semg1, sema1, semb1)
        return carry

    lax.fori_loop(0, NPAIR, body, 0)
    plsc.subcore_barrier()

    pltpu.sync_copy(g_sh.at[pl.ds(r0, R_MAIN)],
                    g_out.at[c, pl.ds(r0, R_MAIN)])
    pltpu.sync_copy(es_sh.at[pl.ds(r0, R_MAIN)],
                    es_out.at[c, pl.ds(r0, R_MAIN)])
    pltpu.sync_copy(deg_sh.at[pl.ds(r0, R_MAIN)],
                    deg_out.at[c, pl.ds(r0, R_MAIN)])

    @pl.when(s == 0)
    def _():
        pltpu.sync_copy(g_sh.at[pl.ds(t0, R_TAIL)],
                        g_out.at[c, pl.ds(t0, R_TAIL)])
        pltpu.sync_copy(es_sh.at[pl.ds(t0, R_TAIL)],
                        es_out.at[c, pl.ds(t0, R_TAIL)])
        pltpu.sync_copy(deg_sh.at[pl.ds(t0, R_TAIL)],
                        deg_out.at[c, pl.ds(t0, R_TAIL)])


def _sc_fused(node2, edge_in, eidx, z64, z16, ones16, ebase_wide, hu2, hv2):
    mesh = plsc.VectorSubcoreMesh(core_axis_name="c", subcore_axis_name="s")
    f32 = jnp.float32
    i32 = jnp.int32
    return pl.kernel(
        _sc_fused_body,
        out_type=(
            jax.ShapeDtypeStruct((NC, N_NODES, 64), f32),
            jax.ShapeDtypeStruct((NC, N_NODES, 16), f32),
            jax.ShapeDtypeStruct((NC, N_NODES, 16), f32),
            jax.ShapeDtypeStruct((N_EDGES, 16), f32),
        ),
        mesh=mesh,
        compiler_params=pltpu.CompilerParams(use_tc_tiling_on_sc=False),
        scratch_types=[
            pltpu.VMEM_SHARED((N_NODES, 64), f32),
            pltpu.VMEM_SHARED((N_NODES, 16), f32),
            pltpu.VMEM_SHARED((N_NODES, 16), f32),
            pltpu.VMEM((EBLK,), i32),
            pltpu.VMEM((EBLK,), i32),
            pltpu.VMEM((EBLK,), i32),
            pltpu.VMEM((EBLK,), i32),
            pltpu.VMEM((EBLK,), i32),
            pltpu.VMEM((EBLK,), i32),
            pltpu.VMEM((EBLK, 64), f32),
            pltpu.VMEM((EBLK, 64), f32),
            pltpu.VMEM((EBLK, 16), f32),
            pltpu.VMEM((EBLK, 16), f32),
            pltpu.VMEM((EBLK, 16), f32),
            pltpu.VMEM((EBLK, 16), f32),
            pltpu.VMEM((EBLK, 16), f32),
            pltpu.VMEM((EBLK, 16), f32),
            pltpu.VMEM((EBLK // 8, 128), f32),
            pltpu.VMEM((EBLK, 16), f32),
            pltpu.SemaphoreType.DMA,
            pltpu.SemaphoreType.DMA,
            pltpu.SemaphoreType.DMA,
            pltpu.SemaphoreType.DMA,
            pltpu.SemaphoreType.DMA,
            pltpu.SemaphoreType.DMA,
        ],
    )(node2, edge_in, eidx, z64, z16, ones16, ebase_wide, hu2, hv2)


def _tc_pre_body(skip_ref, node_ref, edge_ref, wu_ref, wv_ref, we_ref,
                 wbig_ref, bbig_ref, hu2_ref, hv2_ref, ebase_ref):
    f32 = jnp.float32
    node = node_ref[...]
    tu = jnp.dot(node, wu_ref[...], preferred_element_type=f32)
    tv = (jnp.dot(skip_ref[...], wv_ref[0:128], preferred_element_type=f32)
          + jnp.dot(node, wv_ref[128:256], preferred_element_type=f32))
    w2 = we_ref[16:32]
    hu2_ref[...] = jnp.dot(tu, w2, preferred_element_type=f32)
    hv2_ref[...] = jnp.dot(tv, w2, preferred_element_type=f32)
    ebase_ref[...] = (jnp.dot(edge_ref[...], wbig_ref[...],
                              preferred_element_type=f32) + bbig_ref[...])


def _tc_pre(skip, node_in, edge_wide, w_n2e_u, w_n2e_v, w_e2e, w_big,
            bias_big):
    f32 = jnp.float32
    nb = 1000
    grid = N_NODES // nb
    ewb = edge_wide.shape[0] // grid
    full = lambda a: pl.BlockSpec(a.shape, lambda i: (0,) * a.ndim)
    return pl.pallas_call(
        _tc_pre_body,
        grid=(grid,),
        in_specs=[
            pl.BlockSpec((nb, 128), lambda i: (i, 0)),
            pl.BlockSpec((nb, 128), lambda i: (i, 0)),
            pl.BlockSpec((ewb, 128), lambda i: (i, 0)),
            full(w_n2e_u),
            full(w_n2e_v),
            full(w_e2e),
            full(w_big),
            pl.BlockSpec((1, 128), lambda i: (0, 0)),
        ],
        out_specs=[
            pl.BlockSpec((nb, 16), lambda i: (i, 0)),
            pl.BlockSpec((nb, 16), lambda i: (i, 0)),
            pl.BlockSpec((ewb, 128), lambda i: (i, 0)),
        ],
        out_shape=[
            jax.ShapeDtypeStruct((N_NODES, 16), f32),
            jax.ShapeDtypeStruct((N_NODES, 16), f32),
            jax.ShapeDtypeStruct((N_EDGES // 8, 128), f32),
        ],
    )(skip, node_in, edge_wide, w_n2e_u, w_n2e_v, w_e2e, w_big,
      bias_big.reshape(1, 128))


def _tc_post_body(skip_ref, node_ref, g_ref, es_ref, deg_ref,
                  wu_ref, wen_ref, wv_ref, bn_ref, out_ref):
    f32 = jnp.float32
    msg = (jnp.dot(g_ref[0], wu_ref[0:64], preferred_element_type=f32)
           + jnp.dot(g_ref[1], wu_ref[64:128], preferred_element_type=f32)
           + jnp.dot(es_ref[0] + es_ref[1], wen_ref[...],
                     preferred_element_type=f32))
    deg = deg_ref[0, :, 0:1] + deg_ref[1, :, 0:1]
    hn = msg / jnp.maximum(deg, 1.0)
    out_ref[...] = (
        jnp.dot(skip_ref[...], wv_ref[0:128], preferred_element_type=f32)
        + jnp.dot(node_ref[...], wv_ref[128:256], preferred_element_type=f32)
        + jnp.dot(hn, wv_ref[256:384], preferred_element_type=f32)
        + bn_ref[...])


def _tc_post(skip, node_in, g_p, es_p, deg_p, w_n2n_u, w_e2n, w_n2n_v,
             bias_n):
    f32 = jnp.float32
    nb = 1000
    grid = N_NODES // nb
    full = lambda a: pl.BlockSpec(a.shape, lambda i: (0,) * a.ndim)
    return pl.pallas_call(
        _tc_post_body,
        grid=(grid,),
        in_specs=[
            pl.BlockSpec((nb, 128), lambda i: (i, 0)),
            pl.BlockSpec((nb, 128), lambda i: (i, 0)),
            pl.BlockSpec((NC, nb, 64), lambda i: (0, i, 0)),
            pl.BlockSpec((NC, nb, 16), lambda i: (0, i, 0)),
            pl.BlockSpec((NC, nb, 16), lambda i: (0, i, 0)),
            full(w_n2n_u),
            full(w_e2n),
            full(w_n2n_v),
            pl.BlockSpec((1, 128), lambda i: (0, 0)),
        ],
        out_specs=pl.BlockSpec((nb, 128), lambda i: (i, 0)),
        out_shape=jax.ShapeDtypeStruct((N_NODES, 128), f32),
    )(skip, node_in, g_p, es_p, deg_p, w_n2n_u, w_e2n, w_n2n_v,
      bias_n.reshape(1, 128))


@jax.jit
def _run(Skipnode_in_feats, node_in_feats, edge_in_feats, edge_index,
         weight_n2n_u, weight_n2n_v, weight_e2n, bias_n,
         weight_n2e_u, weight_n2e_v, weight_e2e, bias_e):
    f32 = jnp.float32
    eidx = edge_index.astype(jnp.int32)
    z64 = jnp.zeros((R_MAIN, 64), f32)
    z16 = jnp.zeros((R_MAIN, 16), f32)
    ones16 = jnp.ones((EBLK, 16), f32)
    node2 = node_in_feats.reshape(2 * N_NODES, 64)

    edge_wide = edge_in_feats.reshape(N_EDGES // 8, 128)
    w_big = jnp.kron(jnp.eye(8, dtype=f32), weight_e2e[:16])
    bias_big = jnp.tile(bias_e, 8)
    hu2, hv2, ebase_wide = _tc_pre(Skipnode_in_feats, node_in_feats,
                                   edge_wide, weight_n2e_u, weight_n2e_v,
                                   weight_e2e, w_big, bias_big)
    g_p, es_p, deg_p, e_out = _sc_fused(node2, edge_in_feats, eidx,
                                        z64, z16, ones16, ebase_wide,
                                        hu2, hv2)
    h_out = _tc_post(Skipnode_in_feats, node_in_feats, g_p, es_p, deg_p,
                     weight_n2n_u, weight_e2n, weight_n2n_v, bias_n)
    return h_out, e_out


def kernel(Skipnode_in_feats, node_in_feats, edge_in_feats, edge_index,
           weight_n2n_u, weight_n2n_v, weight_e2n, bias_n,
           weight_n2e_u, weight_n2e_v, weight_e2e, bias_e):
    return _run(Skipnode_in_feats, node_in_feats, edge_in_feats, edge_index,
                weight_n2n_u, weight_n2n_v, weight_e2n, bias_n,
                weight_n2e_u, weight_n2e_v, weight_e2e, bias_e)


# speedup vs baseline: 1.1846x; 1.1846x over previous
import functools

import jax
import jax.numpy as jnp
from jax import lax
from jax.experimental import pallas as pl
from jax.experimental.pallas import tpu as pltpu
from jax.experimental.pallas import tpu_sc as plsc

N_NODES = 10000
N_EDGES = 320000
EBLK = 128
NBLK = N_EDGES // EBLK
NC, NS = 2, 16
NPAIR = (NBLK // NS + 1) // 2 + 1
NPAIR_C = (NBLK // (NC * NS) + 1) // 2 + 1
R_MAIN = (N_NODES // NS) // 8 * 8
R_TAIL = N_NODES - NS * R_MAIN


def _sc_scatter_body(node2_hbm, edge_hbm, eidx_hbm, z64_hbm,
                     z16_hbm, ones_hbm, g_out, es_out, deg_out,
                     g_sh, es_sh, deg_sh,
                     sidx0, sidx1, gidx0, gidx1, didx0, didx1,
                     rows0, rows1, erows_v, ones_v, sem0, sem1):
    c = lax.axis_index("c")
    s = lax.axis_index("s")
    r0 = s * R_MAIN
    t0 = NS * R_MAIN

    pltpu.sync_copy(z64_hbm, g_sh.at[pl.ds(r0, R_MAIN)])
    pltpu.sync_copy(z16_hbm, es_sh.at[pl.ds(r0, R_MAIN)])
    pltpu.sync_copy(z16_hbm, deg_sh.at[pl.ds(r0, R_MAIN)])
    pltpu.sync_copy(ones_hbm, ones_v)

    @pl.when(s == 0)
    def _():
        pltpu.sync_copy(z64_hbm.at[pl.ds(0, R_TAIL)],
                        g_sh.at[pl.ds(t0, R_TAIL)])
        pltpu.sync_copy(z16_hbm.at[pl.ds(0, R_TAIL)],
                        es_sh.at[pl.ds(t0, R_TAIL)])
        pltpu.sync_copy(z16_hbm.at[pl.ds(0, R_TAIL)],
                        deg_sh.at[pl.ds(t0, R_TAIL)])

    plsc.subcore_barrier()

    def issue(tb, sidx, gidx, didx, rows, sem):
        j = tb * NS + s

        @pl.when(j < NBLK)
        def _():
            off = j * EBLK
            pltpu.sync_copy(eidx_hbm.at[0, pl.ds(off, EBLK)], sidx)
            pltpu.sync_copy(eidx_hbm.at[1, pl.ds(off, EBLK)], didx)
            for k in range(EBLK // 16):
                sl = pl.ds(k * 16, 16)
                gidx[sl] = sidx[sl] * 2 + c
            pltpu.make_async_copy(node2_hbm.at[gidx], rows, sem).start()

    def process(tb, gidx, didx, rows, sem):
        j = tb * NS + s

        @pl.when(j < NBLK)
        def _():
            off = j * EBLK
            pltpu.make_async_copy(node2_hbm.at[gidx], rows, sem).wait()
            pltpu.sync_copy(rows, g_sh.at[didx], add=True)

            @pl.when(c == 0)
            def _():
                pltpu.sync_copy(edge_hbm.at[pl.ds(off, EBLK)], erows_v)
                pltpu.sync_copy(erows_v, es_sh.at[didx], add=True)

            @pl.when(c == 1)
            def _():
                pltpu.sync_copy(ones_v, deg_sh.at[didx], add=True)

    issue(0, sidx0, gidx0, didx0, rows0, sem0)

    def body(t, carry):
        tb = t * 2
        issue(tb + 1, sidx1, gidx1, didx1, rows1, sem1)
        process(tb, gidx0, didx0, rows0, sem0)
        issue(tb + 2, sidx0, gidx0, didx0, rows0, sem0)
        process(tb + 1, gidx1, didx1, rows1, sem1)
        return carry

    lax.fori_loop(0, NPAIR, body, 0)
    plsc.subcore_barrier()

    pltpu.sync_copy(g_sh.at[pl.ds(r0, R_MAIN)],
                    g_out.at[c, pl.ds(r0, R_MAIN)])
    pltpu.sync_copy(es_sh.at[pl.ds(r0, R_MAIN)],
                    es_out.at[c, pl.ds(r0, R_MAIN)])
    pltpu.sync_copy(deg_sh.at[pl.ds(r0, R_MAIN)],
                    deg_out.at[c, pl.ds(r0, R_MAIN)])

    @pl.when(s == 0)
    def _():
        pltpu.sync_copy(g_sh.at[pl.ds(t0, R_TAIL)],
                        g_out.at[c, pl.ds(t0, R_TAIL)])
        pltpu.sync_copy(es_sh.at[pl.ds(t0, R_TAIL)],
                        es_out.at[c, pl.ds(t0, R_TAIL)])
        pltpu.sync_copy(deg_sh.at[pl.ds(t0, R_TAIL)],
                        deg_out.at[c, pl.ds(t0, R_TAIL)])


def _sc_scatter(node2, edge_in, eidx, z64, z16, ones16):
    mesh = plsc.VectorSubcoreMesh(core_axis_name="c", subcore_axis_name="s")
    f32 = jnp.float32
    i32 = jnp.int32
    return pl.kernel(
        _sc_scatter_body,
        out_type=(
            jax.ShapeDtypeStruct((NC, N_NODES, 64), f32),
            jax.ShapeDtypeStruct((NC, N_NODES, 16), f32),
            jax.ShapeDtypeStruct((NC, N_NODES, 16), f32),
        ),
        mesh=mesh,
        compiler_params=pltpu.CompilerParams(use_tc_tiling_on_sc=False),
        scratch_types=[
            pltpu.VMEM_SHARED((N_NODES, 64), f32),
            pltpu.VMEM_SHARED((N_NODES, 16), f32),
            pltpu.VMEM_SHARED((N_NODES, 16), f32),
            pltpu.VMEM((EBLK,), i32),
            pltpu.VMEM((EBLK,), i32),
            pltpu.VMEM((EBLK,), i32),
            pltpu.VMEM((EBLK,), i32),
            pltpu.VMEM((EBLK,), i32),
            pltpu.VMEM((EBLK,), i32),
            pltpu.VMEM((EBLK, 64), f32),
            pltpu.VMEM((EBLK, 64), f32),
            pltpu.VMEM((EBLK, 16), f32),
            pltpu.VMEM((EBLK, 16), f32),
            pltpu.SemaphoreType.DMA,
            pltpu.SemaphoreType.DMA,
        ],
    )(node2, edge_in, eidx, z64, z16, ones16)


def _sc_edge_body(ebw_hbm, hu2_hbm, hv2_hbm, eidx_hbm, eout_hbm,
                  sidx0, sidx1, didx0, didx1, a0, a1, b0, b1, acc_w, nacc,
                  sema0, sema1, semb0, semb1):
    c = lax.axis_index("c")
    s = lax.axis_index("s")
    w = s * NC + c
    WROWS = EBLK // 8

    def issue(tb, sidx, didx, a_v, b_v, sem_a, sem_b):
        j = tb * (NC * NS) + w

        @pl.when(j < NBLK)
        def _():
            off = j * EBLK
            pltpu.sync_copy(eidx_hbm.at[0, pl.ds(off, EBLK)], sidx)
            pltpu.sync_copy(eidx_hbm.at[1, pl.ds(off, EBLK)], didx)
            pltpu.make_async_copy(hu2_hbm.at[sidx], a_v, sem_a).start()
            pltpu.make_async_copy(hv2_hbm.at[didx], b_v, sem_b).start()

    def process(tb, sidx, didx, a_v, b_v, sem_a, sem_b):
        j = tb * (NC * NS) + w

        @pl.when(j < NBLK)
        def _():
            wr = j * WROWS
            pltpu.sync_copy(ebw_hbm.at[pl.ds(wr, WROWS)], acc_w)
            pltpu.make_async_copy(hu2_hbm.at[sidx], a_v, sem_a).wait()
            pltpu.make_async_copy(hv2_hbm.at[didx], b_v, sem_b).wait()

            def add_rows(q, carry2):
                for u in range(8):
                    r = q * 8 + u
                    sl = pl.ds(u * 16, 16)
                    nacc[r] = acc_w[q, sl] + a_v[r] + b_v[r]
                return carry2

            lax.fori_loop(0, WROWS, add_rows, 0)
            pltpu.sync_copy(nacc, eout_hbm.at[pl.ds(j * EBLK, EBLK)])

    issue(0, sidx0, didx0, a0, b0, sema0, semb0)

    def body(t, carry):
        tb = t * 2
        issue(tb + 1, sidx1, didx1, a1, b1, sema1, semb1)
        process(tb, sidx0, didx0, a0, b0, sema0, semb0)
        issue(tb + 2, sidx0, didx0, a0, b0, sema0, semb0)
        process(tb + 1, sidx1, didx1, a1, b1, sema1, semb1)
        return carry

    lax.fori_loop(0, NPAIR_C, body, 0)


def _sc_edge(ebase_wide, hu2, hv2, eidx):
    mesh = plsc.VectorSubcoreMesh(core_axis_name="c", subcore_axis_name="s")
    f32 = jnp.float32
    i32 = jnp.int32
    return pl.kernel(
        _sc_edge_body,
        out_type=jax.ShapeDtypeStruct((N_EDGES, 16), f32),
        mesh=mesh,
        compiler_params=pltpu.CompilerParams(use_tc_tiling_on_sc=False),
        scratch_types=[
            pltpu.VMEM((EBLK,), i32),
            pltpu.VMEM((EBLK,), i32),
            pltpu.VMEM((EBLK,), i32),
            pltpu.VMEM((EBLK,), i32),
            pltpu.VMEM((EBLK, 16), f32),
            pltpu.VMEM((EBLK, 16), f32),
            pltpu.VMEM((EBLK, 16), f32),
            pltpu.VMEM((EBLK, 16), f32),
            pltpu.VMEM((EBLK // 8, 128), f32),
            pltpu.VMEM((EBLK, 16), f32),
            pltpu.SemaphoreType.DMA,
            pltpu.SemaphoreType.DMA,
            pltpu.SemaphoreType.DMA,
            pltpu.SemaphoreType.DMA,
        ],
    )(ebase_wide, hu2, hv2, eidx)


def _tc_pre_body(skip_ref, node_ref, edge_ref, wu_ref, wv_ref, we_ref,
                 wbig_ref, bbig_ref, hu2_ref, hv2_ref, ebase_ref):
    f32 = jnp.float32
    node = node_ref[...]
    tu = jnp.dot(node, wu_ref[...], preferred_element_type=f32)
    tv = (jnp.dot(skip_ref[...], wv_ref[0:128], preferred_element_type=f32)
          + jnp.dot(node, wv_ref[128:256], preferred_element_type=f32))
    w2 = we_ref[16:32]
    hu2_ref[...] = jnp.dot(tu, w2, preferred_element_type=f32)
    hv2_ref[...] = jnp.dot(tv, w2, preferred_element_type=f32)
    ebase_ref[...] = (jnp.dot(edge_ref[...], wbig_ref[...],
                              preferred_element_type=f32) + bbig_ref[...])


def _tc_pre(skip, node_in, edge_wide, w_n2e_u, w_n2e_v, w_e2e, w_big,
            bias_big):
    f32 = jnp.float32
    nb = 1000
    grid = N_NODES // nb
    ewb = edge_wide.shape[0] // grid
    full = lambda a: pl.BlockSpec(a.shape, lambda i: (0,) * a.ndim)
    return pl.pallas_call(
        _tc_pre_body,
        grid=(grid,),
        in_specs=[
            pl.BlockSpec((nb, 128), lambda i: (i, 0)),
            pl.BlockSpec((nb, 128), lambda i: (i, 0)),
            pl.BlockSpec((ewb, 128), lambda i: (i, 0)),
            full(w_n2e_u),
            full(w_n2e_v),
            full(w_e2e),
            full(w_big),
            pl.BlockSpec((1, 128), lambda i: (0, 0)),
        ],
        out_specs=[
            pl.BlockSpec((nb, 16), lambda i: (i, 0)),
            pl.BlockSpec((nb, 16), lambda i: (i, 0)),
            pl.BlockSpec((ewb, 128), lambda i: (i, 0)),
        ],
        out_shape=[
            jax.ShapeDtypeStruct((N_NODES, 16), f32),
            jax.ShapeDtypeStruct((N_NODES, 16), f32),
            jax.ShapeDtypeStruct((N_EDGES // 8, 128), f32),
        ],
    )(skip, node_in, edge_wide, w_n2e_u, w_n2e_v, w_e2e, w_big,
      bias_big.reshape(1, 128))


def _tc_post_body(skip_ref, node_ref, g_ref, es_ref, deg_ref,
                  wu_ref, wen_ref, wv_ref, bn_ref, out_ref):
    f32 = jnp.float32
    msg = (jnp.dot(g_ref[0], wu_ref[0:64], preferred_element_type=f32)
           + jnp.dot(g_ref[1], wu_ref[64:128], preferred_element_type=f32)
           + jnp.dot(es_ref[0] + es_ref[1], wen_ref[...],
                     preferred_element_type=f32))
    deg = deg_ref[0, :, 0:1] + deg_ref[1, :, 0:1]
    hn = msg / jnp.maximum(deg, 1.0)
    out_ref[...] = (
        jnp.dot(skip_ref[...], wv_ref[0:128], preferred_element_type=f32)
        + jnp.dot(node_ref[...], wv_ref[128:256], preferred_element_type=f32)
        + jnp.dot(hn, wv_ref[256:384], preferred_element_type=f32)
        + bn_ref[...])


def _tc_post(skip, node_in, g_p, es_p, deg_p, w_n2n_u, w_e2n, w_n2n_v,
             bias_n):
    f32 = jnp.float32
    nb = 1000
    grid = N_NODES // nb
    full = lambda a: pl.BlockSpec(a.shape, lambda i: (0,) * a.ndim)
    return pl.pallas_call(
        _tc_post_body,
        grid=(grid,),
        in_specs=[
            pl.BlockSpec((nb, 128), lambda i: (i, 0)),
            pl.BlockSpec((nb, 128), lambda i: (i, 0)),
            pl.BlockSpec((NC, nb, 64), lambda i: (0, i, 0)),
            pl.BlockSpec((NC, nb, 16), lambda i: (0, i, 0)),
            pl.BlockSpec((NC, nb, 16), lambda i: (0, i, 0)),
            full(w_n2n_u),
            full(w_e2n),
            full(w_n2n_v),
            pl.BlockSpec((1, 128), lambda i: (0, 0)),
        ],
        out_specs=pl.BlockSpec((nb, 128), lambda i: (i, 0)),
        out_shape=jax.ShapeDtypeStruct((N_NODES, 128), f32),
    )(skip, node_in, g_p, es_p, deg_p, w_n2n_u, w_e2n, w_n2n_v,
      bias_n.reshape(1, 128))


@jax.jit
def _run(Skipnode_in_feats, node_in_feats, edge_in_feats, edge_index,
         weight_n2n_u, weight_n2n_v, weight_e2n, bias_n,
         weight_n2e_u, weight_n2e_v, weight_e2e, bias_e):
    f32 = jnp.float32
    eidx = edge_index.astype(jnp.int32)
    z64 = jnp.zeros((R_MAIN, 64), f32)
    z16 = jnp.zeros((R_MAIN, 16), f32)
    ones16 = jnp.ones((EBLK, 16), f32)
    node2 = node_in_feats.reshape(2 * N_NODES, 64)

    edge_wide = edge_in_feats.reshape(N_EDGES // 8, 128)
    w_big = jnp.kron(jnp.eye(8, dtype=f32), weight_e2e[:16])
    bias_big = jnp.tile(bias_e, 8)
    hu2, hv2, ebase_wide = _tc_pre(Skipnode_in_feats, node_in_feats,
                                   edge_wide, weight_n2e_u, weight_n2e_v,
                                   weight_e2e, w_big, bias_big)
    g_p, es_p, deg_p = _sc_scatter(node2, edge_in_feats, eidx,
                                   z64, z16, ones16)
    e_out = _sc_edge(ebase_wide, hu2, hv2, eidx)
    h_out = _tc_post(Skipnode_in_feats, node_in_feats, g_p, es_p, deg_p,
                     weight_n2n_u, weight_e2n, weight_n2n_v, bias_n)
    return h_out, e_out


def kernel(Skipnode_in_feats, node_in_feats, edge_in_feats, edge_index,
           weight_n2n_u, weight_n2n_v, weight_e2n, bias_n,
           weight_n2e_u, weight_n2e_v, weight_e2e, bias_e):
    return _run(Skipnode_in_feats, node_in_feats, edge_in_feats, edge_index,
                weight_n2n_u, weight_n2n_v, weight_e2n, bias_n,
                weight_n2e_u, weight_n2e_v, weight_e2e, bias_e)


# speedup vs baseline: 1.4680x; 1.2392x over previous
import functools

import jax
import jax.numpy as jnp
from jax import lax
from jax.experimental import pallas as pl
from jax.experimental.pallas import tpu as pltpu
from jax.experimental.pallas import tpu_sc as plsc

N_NODES = 10000
N_EDGES = 320000
EBLK = 128
NBLK = N_EDGES // EBLK
NC, NS = 2, 16
NPAIR = (NBLK // NS + 1) // 2 + 1
NPAIR_C = (NBLK // (NC * NS) + 1) // 2 + 1
R_MAIN = (N_NODES // NS) // 8 * 8
R_TAIL = N_NODES - NS * R_MAIN


def _sc_scatter_body(node2_hbm, edge_hbm, eidx_hbm, z64_hbm,
                     z16_hbm, ones_hbm, g_out, es_out, deg_out,
                     g_sh, es_sh, deg_sh,
                     sidx0, sidx1, gidx0, gidx1, didx0, didx1,
                     rows0, rows1, erows_v, ones_v, sem0, sem1):
    c = lax.axis_index("c")
    s = lax.axis_index("s")
    r0 = s * R_MAIN
    t0 = NS * R_MAIN

    pltpu.sync_copy(z64_hbm, g_sh.at[pl.ds(r0, R_MAIN)])
    pltpu.sync_copy(z16_hbm, es_sh.at[pl.ds(r0, R_MAIN)])
    pltpu.sync_copy(z16_hbm, deg_sh.at[pl.ds(r0, R_MAIN)])
    pltpu.sync_copy(ones_hbm, ones_v)

    @pl.when(s == 0)
    def _():
        pltpu.sync_copy(z64_hbm.at[pl.ds(0, R_TAIL)],
                        g_sh.at[pl.ds(t0, R_TAIL)])
        pltpu.sync_copy(z16_hbm.at[pl.ds(0, R_TAIL)],
                        es_sh.at[pl.ds(t0, R_TAIL)])
        pltpu.sync_copy(z16_hbm.at[pl.ds(0, R_TAIL)],
                        deg_sh.at[pl.ds(t0, R_TAIL)])

    plsc.subcore_barrier()

    def issue(tb, sidx, gidx, didx, rows, sem):
        j = tb * NS + s

        @pl.when(j < NBLK)
        def _():
            off = j * EBLK
            pltpu.sync_copy(eidx_hbm.at[0, pl.ds(off, EBLK)], sidx)
            pltpu.sync_copy(eidx_hbm.at[1, pl.ds(off, EBLK)], didx)
            for k in range(EBLK // 16):
                sl = pl.ds(k * 16, 16)
                gidx[sl] = sidx[sl] * 2 + c
            pltpu.make_async_copy(node2_hbm.at[gidx], rows, sem).start()

    def process(tb, gidx, didx, rows, sem):
        j = tb * NS + s

        @pl.when(j < NBLK)
        def _():
            off = j * EBLK
            pltpu.make_async_copy(node2_hbm.at[gidx], rows, sem).wait()
            pltpu.sync_copy(rows, g_sh.at[didx], add=True)

            @pl.when(c == 0)
            def _():
                pltpu.sync_copy(edge_hbm.at[pl.ds(off, EBLK)], erows_v)
                pltpu.sync_copy(erows_v, es_sh.at[didx], add=True)

            @pl.when(c == 1)
            def _():
                pltpu.sync_copy(ones_v, deg_sh.at[didx], add=True)

    issue(0, sidx0, gidx0, didx0, rows0, sem0)

    def body(t, carry):
        tb = t * 2
        issue(tb + 1, sidx1, gidx1, didx1, rows1, sem1)
        process(tb, gidx0, didx0, rows0, sem0)
        issue(tb + 2, sidx0, gidx0, didx0, rows0, sem0)
        process(tb + 1, gidx1, didx1, rows1, sem1)
        return carry

    lax.fori_loop(0, NPAIR, body, 0)
    plsc.subcore_barrier()

    pltpu.sync_copy(g_sh.at[pl.ds(r0, R_MAIN)],
                    g_out.at[c, pl.ds(r0, R_MAIN)])
    pltpu.sync_copy(es_sh.at[pl.ds(r0, R_MAIN)],
                    es_out.at[c, pl.ds(r0, R_MAIN)])
    pltpu.sync_copy(deg_sh.at[pl.ds(r0, R_MAIN)],
                    deg_out.at[c, pl.ds(r0, R_MAIN)])

    @pl.when(s == 0)
    def _():
        pltpu.sync_copy(g_sh.at[pl.ds(t0, R_TAIL)],
                        g_out.at[c, pl.ds(t0, R_TAIL)])
        pltpu.sync_copy(es_sh.at[pl.ds(t0, R_TAIL)],
                        es_out.at[c, pl.ds(t0, R_TAIL)])
        pltpu.sync_copy(deg_sh.at[pl.ds(t0, R_TAIL)],
                        deg_out.at[c, pl.ds(t0, R_TAIL)])


def _sc_scatter(node2, edge_in, eidx, z64, z16, ones16):
    mesh = plsc.VectorSubcoreMesh(core_axis_name="c", subcore_axis_name="s")
    f32 = jnp.float32
    i32 = jnp.int32
    return pl.kernel(
        _sc_scatter_body,
        out_type=(
            jax.ShapeDtypeStruct((NC, N_NODES, 64), f32),
            jax.ShapeDtypeStruct((NC, N_NODES, 16), f32),
            jax.ShapeDtypeStruct((NC, N_NODES, 16), f32),
        ),
        mesh=mesh,
        compiler_params=pltpu.CompilerParams(use_tc_tiling_on_sc=False),
        scratch_types=[
            pltpu.VMEM_SHARED((N_NODES, 64), f32),
            pltpu.VMEM_SHARED((N_NODES, 16), f32),
            pltpu.VMEM_SHARED((N_NODES, 16), f32),
            pltpu.VMEM((EBLK,), i32),
            pltpu.VMEM((EBLK,), i32),
            pltpu.VMEM((EBLK,), i32),
            pltpu.VMEM((EBLK,), i32),
            pltpu.VMEM((EBLK,), i32),
            pltpu.VMEM((EBLK,), i32),
            pltpu.VMEM((EBLK, 64), f32),
            pltpu.VMEM((EBLK, 64), f32),
            pltpu.VMEM((EBLK, 16), f32),
            pltpu.VMEM((EBLK, 16), f32),
            pltpu.SemaphoreType.DMA,
            pltpu.SemaphoreType.DMA,
        ],
    )(node2, edge_in, eidx, z64, z16, ones16)


def _sc_edge_body(hu2_hbm, hv2_hbm, eidx_hbm, eout_hbm,
                  sidx0, sidx1, didx0, didx1, a0, a1, b0, b1, nacc,
                  sema0, sema1, semb0, semb1):
    c = lax.axis_index("c")
    s = lax.axis_index("s")
    w = s * NC + c
    WROWS = EBLK // 8

    def issue(tb, sidx, didx, a_v, b_v, sem_a, sem_b):
        j = tb * (NC * NS) + w

        @pl.when(j < NBLK)
        def _():
            off = j * EBLK
            pltpu.sync_copy(eidx_hbm.at[0, pl.ds(off, EBLK)], sidx)
            pltpu.sync_copy(eidx_hbm.at[1, pl.ds(off, EBLK)], didx)
            pltpu.make_async_copy(hu2_hbm.at[sidx], a_v, sem_a).start()
            pltpu.make_async_copy(hv2_hbm.at[didx], b_v, sem_b).start()

    def process(tb, sidx, didx, a_v, b_v, sem_a, sem_b):
        j = tb * (NC * NS) + w

        @pl.when(j < NBLK)
        def _():
            pltpu.make_async_copy(hu2_hbm.at[sidx], a_v, sem_a).wait()
            pltpu.make_async_copy(hv2_hbm.at[didx], b_v, sem_b).wait()

            def add_rows(q, carry2):
                for u in range(8):
                    r = q * 8 + u
                    nacc[r] = a_v[r] + b_v[r]
                return carry2

            lax.fori_loop(0, WROWS, add_rows, 0)
            pltpu.sync_copy(nacc, eout_hbm.at[pl.ds(j * EBLK, EBLK)])

    issue(0, sidx0, didx0, a0, b0, sema0, semb0)

    def body(t, carry):
        tb = t * 2
        issue(tb + 1, sidx1, didx1, a1, b1, sema1, semb1)
        process(tb, sidx0, didx0, a0, b0, sema0, semb0)
        issue(tb + 2, sidx0, didx0, a0, b0, sema0, semb0)
        process(tb + 1, sidx1, didx1, a1, b1, sema1, semb1)
        return carry

    lax.fori_loop(0, NPAIR_C, body, 0)


def _sc_edge(hu2, hv2, eidx):
    mesh = plsc.VectorSubcoreMesh(core_axis_name="c", subcore_axis_name="s")
    f32 = jnp.float32
    i32 = jnp.int32
    return pl.kernel(
        _sc_edge_body,
        out_type=jax.ShapeDtypeStruct((N_EDGES, 16), f32),
        mesh=mesh,
        compiler_params=pltpu.CompilerParams(use_tc_tiling_on_sc=False),
        scratch_types=[
            pltpu.VMEM((EBLK,), i32),
            pltpu.VMEM((EBLK,), i32),
            pltpu.VMEM((EBLK,), i32),
            pltpu.VMEM((EBLK,), i32),
            pltpu.VMEM((EBLK, 16), f32),
            pltpu.VMEM((EBLK, 16), f32),
            pltpu.VMEM((EBLK, 16), f32),
            pltpu.VMEM((EBLK, 16), f32),
            pltpu.VMEM((EBLK, 16), f32),
            pltpu.SemaphoreType.DMA,
            pltpu.SemaphoreType.DMA,
            pltpu.SemaphoreType.DMA,
            pltpu.SemaphoreType.DMA,
        ],
    )(hu2, hv2, eidx)


def _tc_pre_body(skip_ref, node_ref, wu_ref, wv_ref, we_ref,
                 be_ref, hu2_ref, hv2_ref):
    f32 = jnp.float32
    node = node_ref[...]
    tu = jnp.dot(node, wu_ref[...], preferred_element_type=f32)
    tv = (jnp.dot(skip_ref[...], wv_ref[0:128], preferred_element_type=f32)
          + jnp.dot(node, wv_ref[128:256], preferred_element_type=f32))
    w2 = we_ref[16:32]
    hu2_ref[...] = jnp.dot(tu, w2, preferred_element_type=f32)
    hv2_ref[...] = jnp.dot(tv, w2, preferred_element_type=f32) + be_ref[...]


def _tc_pre(skip, node_in, w_n2e_u, w_n2e_v, w_e2e, bias_e):
    f32 = jnp.float32
    nb = 1000
    grid = N_NODES // nb
    full = lambda a: pl.BlockSpec(a.shape, lambda i: (0,) * a.ndim)
    return pl.pallas_call(
        _tc_pre_body,
        grid=(grid,),
        in_specs=[
            pl.BlockSpec((nb, 128), lambda i: (i, 0)),
            pl.BlockSpec((nb, 128), lambda i: (i, 0)),
            full(w_n2e_u),
            full(w_n2e_v),
            full(w_e2e),
            pl.BlockSpec((1, 16), lambda i: (0, 0)),
        ],
        out_specs=[
            pl.BlockSpec((nb, 16), lambda i: (i, 0)),
            pl.BlockSpec((nb, 16), lambda i: (i, 0)),
        ],
        out_shape=[
            jax.ShapeDtypeStruct((N_NODES, 16), f32),
            jax.ShapeDtypeStruct((N_NODES, 16), f32),
        ],
    )(skip, node_in, w_n2e_u, w_n2e_v, w_e2e, bias_e.reshape(1, 16))


def _tc_edge_body(edge_ref, gsum_ref, wbig_ref, out_ref):
    out_ref[...] = (jnp.dot(edge_ref[...], wbig_ref[...],
                            preferred_element_type=jnp.float32)
                    + gsum_ref[...])


def _tc_edge(edge_wide, gsum_wide, w_big):
    f32 = jnp.float32
    grid = 10
    ewb = edge_wide.shape[0] // grid
    return pl.pallas_call(
        _tc_edge_body,
        grid=(grid,),
        in_specs=[
            pl.BlockSpec((ewb, 128), lambda i: (i, 0)),
            pl.BlockSpec((ewb, 128), lambda i: (i, 0)),
            pl.BlockSpec(w_big.shape, lambda i: (0, 0)),
        ],
        out_specs=pl.BlockSpec((ewb, 128), lambda i: (i, 0)),
        out_shape=jax.ShapeDtypeStruct((N_EDGES // 8, 128), f32),
    )(edge_wide, gsum_wide, w_big)


def _tc_post_body(skip_ref, node_ref, g_ref, es_ref, deg_ref,
                  wu_ref, wen_ref, wv_ref, bn_ref, out_ref):
    f32 = jnp.float32
    msg = (jnp.dot(g_ref[0], wu_ref[0:64], preferred_element_type=f32)
           + jnp.dot(g_ref[1], wu_ref[64:128], preferred_element_type=f32)
           + jnp.dot(es_ref[0] + es_ref[1], wen_ref[...],
                     preferred_element_type=f32))
    deg = deg_ref[0, :, 0:1] + deg_ref[1, :, 0:1]
    hn = msg / jnp.maximum(deg, 1.0)
    out_ref[...] = (
        jnp.dot(skip_ref[...], wv_ref[0:128], preferred_element_type=f32)
        + jnp.dot(node_ref[...], wv_ref[128:256], preferred_element_type=f32)
        + jnp.dot(hn, wv_ref[256:384], preferred_element_type=f32)
        + bn_ref[...])


def _tc_post(skip, node_in, g_p, es_p, deg_p, w_n2n_u, w_e2n, w_n2n_v,
             bias_n):
    f32 = jnp.float32
    nb = 1000
    grid = N_NODES // nb
    full = lambda a: pl.BlockSpec(a.shape, lambda i: (0,) * a.ndim)
    return pl.pallas_call(
        _tc_post_body,
        grid=(grid,),
        in_specs=[
            pl.BlockSpec((nb, 128), lambda i: (i, 0)),
            pl.BlockSpec((nb, 128), lambda i: (i, 0)),
            pl.BlockSpec((NC, nb, 64), lambda i: (0, i, 0)),
            pl.BlockSpec((NC, nb, 16), lambda i: (0, i, 0)),
            pl.BlockSpec((NC, nb, 16), lambda i: (0, i, 0)),
            full(w_n2n_u),
            full(w_e2n),
            full(w_n2n_v),
            pl.BlockSpec((1, 128), lambda i: (0, 0)),
        ],
        out_specs=pl.BlockSpec((nb, 128), lambda i: (i, 0)),
        out_shape=jax.ShapeDtypeStruct((N_NODES, 128), f32),
    )(skip, node_in, g_p, es_p, deg_p, w_n2n_u, w_e2n, w_n2n_v,
      bias_n.reshape(1, 128))


@jax.jit
def _run(Skipnode_in_feats, node_in_feats, edge_in_feats, edge_index,
         weight_n2n_u, weight_n2n_v, weight_e2n, bias_n,
         weight_n2e_u, weight_n2e_v, weight_e2e, bias_e):
    f32 = jnp.float32
    eidx = edge_index.astype(jnp.int32)
    z64 = jnp.zeros((R_MAIN, 64), f32)
    z16 = jnp.zeros((R_MAIN, 16), f32)
    ones16 = jnp.ones((EBLK, 16), f32)
    node2 = node_in_feats.reshape(2 * N_NODES, 64)

    edge_wide = edge_in_feats.reshape(N_EDGES // 8, 128)
    w_big = jnp.kron(jnp.eye(8, dtype=f32), weight_e2e[:16])
    hu2, hv2 = _tc_pre(Skipnode_in_feats, node_in_feats,
                       weight_n2e_u, weight_n2e_v, weight_e2e, bias_e)
    g_p, es_p, deg_p = _sc_scatter(node2, edge_in_feats, eidx,
                                   z64, z16, ones16)
    gsum = _sc_edge(hu2, hv2, eidx)
    eout_wide = _tc_edge(edge_wide, gsum.reshape(N_EDGES // 8, 128), w_big)
    e_out = eout_wide.reshape(N_EDGES, 16)
    h_out = _tc_post(Skipnode_in_feats, node_in_feats, g_p, es_p, deg_p,
                     weight_n2n_u, weight_e2n, weight_n2n_v, bias_n)
    return h_out, e_out


def kernel(Skipnode_in_feats, node_in_feats, edge_in_feats, edge_index,
           weight_n2n_u, weight_n2n_v, weight_e2n, bias_n,
           weight_n2e_u, weight_n2e_v, weight_e2e, bias_e):
    return _run(Skipnode_in_feats, node_in_feats, edge_in_feats, edge_index,
                weight_n2n_u, weight_n2n_v, weight_e2n, bias_n,
                weight_n2e_u, weight_n2e_v, weight_e2e, bias_e)


# speedup vs baseline: 1.4694x; 1.0009x over previous
import functools

import jax
import jax.numpy as jnp
from jax import lax
from jax.experimental import pallas as pl
from jax.experimental.pallas import tpu as pltpu
from jax.experimental.pallas import tpu_sc as plsc

N_NODES = 10000
N_EDGES = 320000
EBLK = 128
NBLK = N_EDGES // EBLK
NC, NS = 2, 16
NPAIR = (NBLK // NS + 1) // 2 + 1
NPAIR_C = (NBLK // (NC * NS) + 1) // 2 + 1
R_MAIN = (N_NODES // NS) // 8 * 8
R_TAIL = N_NODES - NS * R_MAIN


def _sc_scatter_body(node2_hbm, edge_hbm, eidx_hbm, z64_hbm,
                     z16_hbm, ones_hbm, g_out, es_out, deg_out,
                     g_sh, es_sh, deg_sh,
                     sidx0, sidx1, gidx0, gidx1, didx0, didx1,
                     rows0, rows1, erows_v, ones_v, sem0, sem1):
    c = lax.axis_index("c")
    s = lax.axis_index("s")
    r0 = s * R_MAIN
    t0 = NS * R_MAIN

    pltpu.sync_copy(z64_hbm, g_sh.at[pl.ds(r0, R_MAIN)])
    pltpu.sync_copy(z16_hbm, es_sh.at[pl.ds(r0, R_MAIN)])
    pltpu.sync_copy(z16_hbm, deg_sh.at[pl.ds(r0, R_MAIN)])
    pltpu.sync_copy(ones_hbm, ones_v)

    @pl.when(s == 0)
    def _():
        pltpu.sync_copy(z64_hbm.at[pl.ds(0, R_TAIL)],
                        g_sh.at[pl.ds(t0, R_TAIL)])
        pltpu.sync_copy(z16_hbm.at[pl.ds(0, R_TAIL)],
                        es_sh.at[pl.ds(t0, R_TAIL)])
        pltpu.sync_copy(z16_hbm.at[pl.ds(0, R_TAIL)],
                        deg_sh.at[pl.ds(t0, R_TAIL)])

    plsc.subcore_barrier()

    def issue(tb, sidx, gidx, didx, rows, sem):
        j = tb * NS + s

        @pl.when(j < NBLK)
        def _():
            off = j * EBLK
            pltpu.sync_copy(eidx_hbm.at[0, pl.ds(off, EBLK)], sidx)
            pltpu.sync_copy(eidx_hbm.at[1, pl.ds(off, EBLK)], didx)
            for k in range(EBLK // 16):
                sl = pl.ds(k * 16, 16)
                gidx[sl] = sidx[sl] * 2 + c
            pltpu.make_async_copy(node2_hbm.at[gidx], rows, sem).start()

    def process(tb, gidx, didx, rows, sem):
        j = tb * NS + s

        @pl.when(j < NBLK)
        def _():
            off = j * EBLK
            pltpu.make_async_copy(node2_hbm.at[gidx], rows, sem).wait()
            pltpu.sync_copy(rows, g_sh.at[didx], add=True)

            @pl.when(lax.rem(j, 2) == c)
            def _():
                pltpu.sync_copy(edge_hbm.at[pl.ds(off, EBLK)], erows_v)
                pltpu.sync_copy(erows_v, es_sh.at[didx], add=True)

            @pl.when(lax.rem(j, 2) != c)
            def _():
                pltpu.sync_copy(ones_v, deg_sh.at[didx], add=True)

    issue(0, sidx0, gidx0, didx0, rows0, sem0)

    def body(t, carry):
        tb = t * 2
        issue(tb + 1, sidx1, gidx1, didx1, rows1, sem1)
        process(tb, gidx0, didx0, rows0, sem0)
        issue(tb + 2, sidx0, gidx0, didx0, rows0, sem0)
        process(tb + 1, gidx1, didx1, rows1, sem1)
        return carry

    lax.fori_loop(0, NPAIR, body, 0)
    plsc.subcore_barrier()

    pltpu.sync_copy(g_sh.at[pl.ds(r0, R_MAIN)],
                    g_out.at[c, pl.ds(r0, R_MAIN)])
    pltpu.sync_copy(es_sh.at[pl.ds(r0, R_MAIN)],
                    es_out.at[c, pl.ds(r0, R_MAIN)])
    pltpu.sync_copy(deg_sh.at[pl.ds(r0, R_MAIN)],
                    deg_out.at[c, pl.ds(r0, R_MAIN)])

    @pl.when(s == 0)
    def _():
        pltpu.sync_copy(g_sh.at[pl.ds(t0, R_TAIL)],
                        g_out.at[c, pl.ds(t0, R_TAIL)])
        pltpu.sync_copy(es_sh.at[pl.ds(t0, R_TAIL)],
                        es_out.at[c, pl.ds(t0, R_TAIL)])
        pltpu.sync_copy(deg_sh.at[pl.ds(t0, R_TAIL)],
                        deg_out.at[c, pl.ds(t0, R_TAIL)])


def _sc_scatter(node2, edge_in, eidx, z64, z16, ones16):
    mesh = plsc.VectorSubcoreMesh(core_axis_name="c", subcore_axis_name="s")
    f32 = jnp.float32
    i32 = jnp.int32
    return pl.kernel(
        _sc_scatter_body,
        out_type=(
            jax.ShapeDtypeStruct((NC, N_NODES, 64), f32),
            jax.ShapeDtypeStruct((NC, N_NODES, 16), f32),
            jax.ShapeDtypeStruct((NC, N_NODES, 16), f32),
        ),
        mesh=mesh,
        compiler_params=pltpu.CompilerParams(use_tc_tiling_on_sc=False),
        scratch_types=[
            pltpu.VMEM_SHARED((N_NODES, 64), f32),
            pltpu.VMEM_SHARED((N_NODES, 16), f32),
            pltpu.VMEM_SHARED((N_NODES, 16), f32),
            pltpu.VMEM((EBLK,), i32),
            pltpu.VMEM((EBLK,), i32),
            pltpu.VMEM((EBLK,), i32),
            pltpu.VMEM((EBLK,), i32),
            pltpu.VMEM((EBLK,), i32),
            pltpu.VMEM((EBLK,), i32),
            pltpu.VMEM((EBLK, 64), f32),
            pltpu.VMEM((EBLK, 64), f32),
            pltpu.VMEM((EBLK, 16), f32),
            pltpu.VMEM((EBLK, 16), f32),
            pltpu.SemaphoreType.DMA,
            pltpu.SemaphoreType.DMA,
        ],
    )(node2, edge_in, eidx, z64, z16, ones16)


def _sc_edge_body(hu2_hbm, hv2_hbm, eidx_hbm, eout_hbm,
                  sidx0, sidx1, didx0, didx1, a0, a1, b0, b1, nacc,
                  sema0, sema1, semb0, semb1):
    c = lax.axis_index("c")
    s = lax.axis_index("s")
    w = s * NC + c
    WROWS = EBLK // 8

    def issue(tb, sidx, didx, a_v, b_v, sem_a, sem_b):
        j = tb * (NC * NS) + w

        @pl.when(j < NBLK)
        def _():
            off = j * EBLK
            pltpu.sync_copy(eidx_hbm.at[0, pl.ds(off, EBLK)], sidx)
            pltpu.sync_copy(eidx_hbm.at[1, pl.ds(off, EBLK)], didx)
            pltpu.make_async_copy(hu2_hbm.at[sidx], a_v, sem_a).start()
            pltpu.make_async_copy(hv2_hbm.at[didx], b_v, sem_b).start()

    def process(tb, sidx, didx, a_v, b_v, sem_a, sem_b):
        j = tb * (NC * NS) + w

        @pl.when(j < NBLK)
        def _():
            pltpu.make_async_copy(hu2_hbm.at[sidx], a_v, sem_a).wait()
            pltpu.make_async_copy(hv2_hbm.at[didx], b_v, sem_b).wait()

            def add_rows(q, carry2):
                for u in range(8):
                    r = q * 8 + u
                    nacc[q, pl.ds(u * 16, 16)] = a_v[r] + b_v[r]
                return carry2

            lax.fori_loop(0, WROWS, add_rows, 0)
            pltpu.sync_copy(nacc, eout_hbm.at[pl.ds(j * WROWS, WROWS)])

    issue(0, sidx0, didx0, a0, b0, sema0, semb0)

    def body(t, carry):
        tb = t * 2
        issue(tb + 1, sidx1, didx1, a1, b1, sema1, semb1)
        process(tb, sidx0, didx0, a0, b0, sema0, semb0)
        issue(tb + 2, sidx0, didx0, a0, b0, sema0, semb0)
        process(tb + 1, sidx1, didx1, a1, b1, sema1, semb1)
        return carry

    lax.fori_loop(0, NPAIR_C, body, 0)


def _sc_edge(hu2, hv2, eidx):
    mesh = plsc.VectorSubcoreMesh(core_axis_name="c", subcore_axis_name="s")
    f32 = jnp.float32
    i32 = jnp.int32
    return pl.kernel(
        _sc_edge_body,
        out_type=jax.ShapeDtypeStruct((N_EDGES // 8, 128), f32),
        mesh=mesh,
        compiler_params=pltpu.CompilerParams(use_tc_tiling_on_sc=False),
        scratch_types=[
            pltpu.VMEM((EBLK,), i32),
            pltpu.VMEM((EBLK,), i32),
            pltpu.VMEM((EBLK,), i32),
            pltpu.VMEM((EBLK,), i32),
            pltpu.VMEM((EBLK, 16), f32),
            pltpu.VMEM((EBLK, 16), f32),
            pltpu.VMEM((EBLK, 16), f32),
            pltpu.VMEM((EBLK, 16), f32),
            pltpu.VMEM((EBLK // 8, 128), f32),
            pltpu.SemaphoreType.DMA,
            pltpu.SemaphoreType.DMA,
            pltpu.SemaphoreType.DMA,
            pltpu.SemaphoreType.DMA,
        ],
    )(hu2, hv2, eidx)


def _tc_pre_body(skip_ref, node_ref, wu_ref, wv_ref, we_ref,
                 be_ref, hu2_ref, hv2_ref):
    f32 = jnp.float32
    node = node_ref[...]
    tu = jnp.dot(node, wu_ref[...], preferred_element_type=f32)
    tv = (jnp.dot(skip_ref[...], wv_ref[0:128], preferred_element_type=f32)
          + jnp.dot(node, wv_ref[128:256], preferred_element_type=f32))
    w2 = we_ref[16:32]
    hu2_ref[...] = jnp.dot(tu, w2, preferred_element_type=f32)
    hv2_ref[...] = jnp.dot(tv, w2, preferred_element_type=f32) + be_ref[...]


def _tc_pre(skip, node_in, w_n2e_u, w_n2e_v, w_e2e, bias_e):
    f32 = jnp.float32
    nb = 1000
    grid = N_NODES // nb
    full = lambda a: pl.BlockSpec(a.shape, lambda i: (0,) * a.ndim)
    return pl.pallas_call(
        _tc_pre_body,
        grid=(grid,),
        in_specs=[
            pl.BlockSpec((nb, 128), lambda i: (i, 0)),
            pl.BlockSpec((nb, 128), lambda i: (i, 0)),
            full(w_n2e_u),
            full(w_n2e_v),
            full(w_e2e),
            pl.BlockSpec((1, 16), lambda i: (0, 0)),
        ],
        out_specs=[
            pl.BlockSpec((nb, 16), lambda i: (i, 0)),
            pl.BlockSpec((nb, 16), lambda i: (i, 0)),
        ],
        out_shape=[
            jax.ShapeDtypeStruct((N_NODES, 16), f32),
            jax.ShapeDtypeStruct((N_NODES, 16), f32),
        ],
    )(skip, node_in, w_n2e_u, w_n2e_v, w_e2e, bias_e.reshape(1, 16))


def _tc_edge_body(edge_ref, gsum_ref, wbig_ref, out_ref):
    out_ref[...] = (jnp.dot(edge_ref[...], wbig_ref[...],
                            preferred_element_type=jnp.float32)
                    + gsum_ref[...])


def _tc_edge(edge_wide, gsum_wide, w_big):
    f32 = jnp.float32
    grid = 10
    ewb = edge_wide.shape[0] // grid
    return pl.pallas_call(
        _tc_edge_body,
        grid=(grid,),
        in_specs=[
            pl.BlockSpec((ewb, 128), lambda i: (i, 0)),
            pl.BlockSpec((ewb, 128), lambda i: (i, 0)),
            pl.BlockSpec(w_big.shape, lambda i: (0, 0)),
        ],
        out_specs=pl.BlockSpec((ewb, 128), lambda i: (i, 0)),
        out_shape=jax.ShapeDtypeStruct((N_EDGES // 8, 128), f32),
    )(edge_wide, gsum_wide, w_big)


def _tc_post_body(skip_ref, node_ref, g_ref, es_ref, deg_ref,
                  wu_ref, wen_ref, wv_ref, bn_ref, out_ref):
    f32 = jnp.float32
    msg = (jnp.dot(g_ref[0], wu_ref[0:64], preferred_element_type=f32)
           + jnp.dot(g_ref[1], wu_ref[64:128], preferred_element_type=f32)
           + jnp.dot(es_ref[0] + es_ref[1], wen_ref[...],
                     preferred_element_type=f32))
    deg = deg_ref[0, :, 0:1] + deg_ref[1, :, 0:1]
    hn = msg / jnp.maximum(deg, 1.0)
    out_ref[...] = (
        jnp.dot(skip_ref[...], wv_ref[0:128], preferred_element_type=f32)
        + jnp.dot(node_ref[...], wv_ref[128:256], preferred_element_type=f32)
        + jnp.dot(hn, wv_ref[256:384], preferred_element_type=f32)
        + bn_ref[...])


def _tc_post(skip, node_in, g_p, es_p, deg_p, w_n2n_u, w_e2n, w_n2n_v,
             bias_n):
    f32 = jnp.float32
    nb = 1000
    grid = N_NODES // nb
    full = lambda a: pl.BlockSpec(a.shape, lambda i: (0,) * a.ndim)
    return pl.pallas_call(
        _tc_post_body,
        grid=(grid,),
        in_specs=[
            pl.BlockSpec((nb, 128), lambda i: (i, 0)),
            pl.BlockSpec((nb, 128), lambda i: (i, 0)),
            pl.BlockSpec((NC, nb, 64), lambda i: (0, i, 0)),
            pl.BlockSpec((NC, nb, 16), lambda i: (0, i, 0)),
            pl.BlockSpec((NC, nb, 16), lambda i: (0, i, 0)),
            full(w_n2n_u),
            full(w_e2n),
            full(w_n2n_v),
            pl.BlockSpec((1, 128), lambda i: (0, 0)),
        ],
        out_specs=pl.BlockSpec((nb, 128), lambda i: (i, 0)),
        out_shape=jax.ShapeDtypeStruct((N_NODES, 128), f32),
    )(skip, node_in, g_p, es_p, deg_p, w_n2n_u, w_e2n, w_n2n_v,
      bias_n.reshape(1, 128))


@jax.jit
def _run(Skipnode_in_feats, node_in_feats, edge_in_feats, edge_index,
         weight_n2n_u, weight_n2n_v, weight_e2n, bias_n,
         weight_n2e_u, weight_n2e_v, weight_e2e, bias_e):
    f32 = jnp.float32
    eidx = edge_index.astype(jnp.int32)
    z64 = jnp.zeros((R_MAIN, 64), f32)
    z16 = jnp.zeros((R_MAIN, 16), f32)
    ones16 = jnp.ones((EBLK, 16), f32)
    node2 = node_in_feats.reshape(2 * N_NODES, 64)

    edge_wide = edge_in_feats.reshape(N_EDGES // 8, 128)
    w_big = jnp.kron(jnp.eye(8, dtype=f32), weight_e2e[:16])
    hu2, hv2 = _tc_pre(Skipnode_in_feats, node_in_feats,
                       weight_n2e_u, weight_n2e_v, weight_e2e, bias_e)
    g_p, es_p, deg_p = _sc_scatter(node2, edge_in_feats, eidx,
                                   z64, z16, ones16)
    gsum_wide = _sc_edge(hu2, hv2, eidx)
    eout_wide = _tc_edge(edge_wide, gsum_wide, w_big)
    e_out = eout_wide.reshape(N_EDGES, 16)
    h_out = _tc_post(Skipnode_in_feats, node_in_feats, g_p, es_p, deg_p,
                     weight_n2n_u, weight_e2n, weight_n2n_v, bias_n)
    return h_out, e_out


def kernel(Skipnode_in_feats, node_in_feats, edge_in_feats, edge_index,
           weight_n2n_u, weight_n2n_v, weight_e2n, bias_n,
           weight_n2e_u, weight_n2e_v, weight_e2e, bias_e):
    return _run(Skipnode_in_feats, node_in_feats, edge_in_feats, edge_index,
                weight_n2n_u, weight_n2n_v, weight_e2n, bias_n,
                weight_n2e_u, weight_n2e_v, weight_e2e, bias_e)
